# baseline (speedup 1.0000x reference)
import functools

import jax
import jax.numpy as jnp
from jax import lax
from jax.experimental import pallas as pl
from jax.experimental.pallas import tpu as pltpu
from jax.experimental.pallas import tpu_sc as plsc

N = 10000
E = 320000
DIN = 128
DH = 64
DOUT = 10
G = 64

NC, NS = 2, 16
NW = NC * NS
CHUNK = 128
CPT = 80
NCHUNKP = NW * CPT
EPAD = NCHUNKP * CHUNK - E
NPAD = 10240
DSLICE = NPAD // NS
RSLICE = NPAD // NS
NB = 4

_mesh = plsc.VectorSubcoreMesh(core_axis_name="c", subcore_axis_name="s")
_sc_params = pltpu.CompilerParams(use_tc_tiling_on_sc=False)


def _wid(cid, sid):
  return sid * NC + cid


@functools.partial(
    pl.kernel,
    out_type=jax.ShapeDtypeStruct((NC * NPAD,), jnp.float32),
    mesh=_mesh,
    scratch_types=[
        pltpu.VMEM((CPT, CHUNK), jnp.int32),
        pltpu.VMEM((CHUNK,), jnp.float32),
        pltpu.VMEM((DSLICE,), jnp.float32),
        pltpu.VMEM_SHARED((NPAD,), jnp.float32),
        pltpu.SemaphoreType.DMA,
    ],
    compiler_params=_sc_params,
)
def _deg(dst_hbm, out_hbm, didx_v, ones_v, zbuf_v, acc_sh, sem):
  cid = lax.axis_index("c")
  sid = lax.axis_index("s")
  wid = _wid(cid, sid)
  pltpu.sync_copy(dst_hbm.at[pl.ds(wid * CPT, CPT)], didx_v)
  for j in range(CHUNK // 16):
    ones_v[pl.ds(j * 16, 16)] = jnp.ones((16,), jnp.float32)
  for j in range(DSLICE // 16):
    zbuf_v[pl.ds(j * 16, 16)] = jnp.zeros((16,), jnp.float32)
  pltpu.sync_copy(zbuf_v, acc_sh.at[pl.ds(sid * DSLICE, DSLICE)])
  plsc.subcore_barrier()

  K = 20
  def group(g, carry):
    def fire(j, c):
      pltpu.async_copy(ones_v, acc_sh.at[didx_v.at[g * K + j]], sem, add=True)
      return c

    lax.fori_loop(0, K, fire, 0)

    def drain(j, c):
      pltpu.make_async_copy(ones_v, acc_sh.at[didx_v.at[g * K + j]],
                            sem).wait()
      return c

    lax.fori_loop(0, K, drain, 0)
    return carry

  lax.fori_loop(0, CPT // K, group, 0)
  plsc.subcore_barrier()
  pltpu.sync_copy(
      acc_sh.at[pl.ds(sid * DSLICE, DSLICE)],
      out_hbm.at[pl.ds(cid * NPAD + sid * DSLICE, DSLICE)],
  )


@functools.partial(
    pl.kernel,
    out_type=jax.ShapeDtypeStruct((NC * NPAD, DH), jnp.float32),
    mesh=_mesh,
    scratch_types=[
    ] + [pltpu.VMEM((CHUNK,), jnp.int32)] * (2 * NB)
      + [pltpu.VMEM((CHUNK, DH), jnp.float32)] * NB + [
        pltpu.VMEM_SHARED((NPAD, DH), jnp.float32),
    ] + [pltpu.SemaphoreType.DMA] * (3 * NB),
    compiler_params=_sc_params,
)
def _agg(h_hbm, src_hbm, dst_hbm, zero_hbm, out_hbm, *rest):
  sidx = rest[:NB]
  didx = rest[NB:2 * NB]
  rows = rest[2 * NB:3 * NB]
  acc_sh = rest[3 * NB]
  isem = rest[3 * NB + 1:3 * NB + 1 + NB]
  gsem = rest[3 * NB + 1 + NB:3 * NB + 1 + 2 * NB]
  ssem = rest[3 * NB + 1 + 2 * NB:]
  cid = lax.axis_index("c")
  sid = lax.axis_index("s")
  wid = _wid(cid, sid)
  base = wid * CPT
  pltpu.sync_copy(zero_hbm.at[pl.ds(sid * RSLICE, RSLICE)],
                  acc_sh.at[pl.ds(sid * RSLICE, RSLICE)])
  plsc.subcore_barrier()

  def i_fire(j, b):
    c = wid + j * NW
    pltpu.async_copy(src_hbm.at[c], sidx[b], isem[b])
    pltpu.async_copy(dst_hbm.at[c], didx[b], isem[b])

  def i_wait(j, b):
    c = wid + j * NW
    pltpu.make_async_copy(src_hbm.at[c], sidx[b], isem[b]).wait()
    pltpu.make_async_copy(dst_hbm.at[c], didx[b], isem[b]).wait()

  def g_fire(b):
    pltpu.async_copy(h_hbm.at[sidx[b]], rows[b], gsem[b])

  def g_wait(b):
    pltpu.make_async_copy(h_hbm.at[sidx[b]], rows[b], gsem[b]).wait()

  def s_fire(b):
    pltpu.async_copy(rows[b], acc_sh.at[didx[b]], ssem[b], add=True)

  def s_wait(b):
    pltpu.make_async_copy(rows[b], acc_sh.at[didx[b]], ssem[b]).wait()

  i_fire(0, 0)
  i_fire(1, 1)
  i_wait(0, 0)
  g_fire(0)

  def group(g, carry):
    for b in range(NB):
      j = g * NB + b
      bn = (b + 1) % NB
      b2 = (b + 2) % NB
      g_wait(b)
      s_fire(b)
      j2 = j + 2

      @pl.when(j2 < CPT)
      def _():
        @pl.when(j2 >= NB)
        def _():
          s_wait(b2)
        i_fire(j2, b2)

      jn = j + 1

      @pl.when(jn < CPT)
      def _():
        i_wait(jn, bn)
        g_fire(bn)

    return carry

  lax.fori_loop(0, CPT // NB, group, 0)
  for b in range(NB):
    s_wait(b)
  plsc.subcore_barrier()
  pltpu.sync_copy(acc_sh.at[pl.ds(sid * RSLICE, RSLICE)],
                  out_hbm.at[pl.ds(cid * NPAD + sid * RSLICE, RSLICE)])


_R = 1000
_GRID = N // _R

_HI = lax.Precision.HIGHEST


def _tc1_body(x_ref, w1_ref, degA_ref, degB_ref, ht_ref, dinv_ref):
  deg = degA_ref[...] + degB_ref[...] + 1.0
  dinv = lax.rsqrt(deg)
  dinv_ref[...] = dinv
  h = jnp.dot(x_ref[...], w1_ref[...], precision=_HI,
              preferred_element_type=jnp.float32)
  ht_ref[...] = h * dinv


def _tc1(x, w1, degA, degB):
  return pl.pallas_call(
      _tc1_body,
      grid=(_GRID,),
      in_specs=[
          pl.BlockSpec((_R, DIN), lambda i: (i, 0)),
          pl.BlockSpec((DIN, DH), lambda i: (0, 0)),
          pl.BlockSpec((_R, 1), lambda i: (i, 0)),
          pl.BlockSpec((_R, 1), lambda i: (i, 0)),
      ],
      out_specs=[
          pl.BlockSpec((_R, DH), lambda i: (i, 0)),
          pl.BlockSpec((_R, 1), lambda i: (i, 0)),
      ],
      out_shape=[
          jax.ShapeDtypeStruct((N, DH), jnp.float32),
          jax.ShapeDtypeStruct((N, 1), jnp.float32),
      ],
  )(x, w1, degA, degB)


def _tc2_body(aggA_ref, aggB_ref, ht1_ref, dinv_ref, w2_ref, b1_ref, ht2_ref):
  agg = aggA_ref[...] + aggB_ref[...] + ht1_ref[...]
  out1 = dinv_ref[...] * agg + b1_ref[...]
  h2 = jnp.dot(out1, w2_ref[...], precision=_HI,
               preferred_element_type=jnp.float32)
  ht2_ref[...] = dinv_ref[...] * h2


def _tc2(aggA, aggB, ht1, dinv, w2, b1):
  return pl.pallas_call(
      _tc2_body,
      grid=(_GRID,),
      in_specs=[
          pl.BlockSpec((_R, DH), lambda i: (i, 0)),
          pl.BlockSpec((_R, DH), lambda i: (i, 0)),
          pl.BlockSpec((_R, DH), lambda i: (i, 0)),
          pl.BlockSpec((_R, 1), lambda i: (i, 0)),
          pl.BlockSpec((DH, DH), lambda i: (0, 0)),
          pl.BlockSpec((1, DH), lambda i: (0, 0)),
      ],
      out_specs=pl.BlockSpec((_R, DH), lambda i: (i, 0)),
      out_shape=jax.ShapeDtypeStruct((N, DH), jnp.float32),
  )(aggA, aggB, ht1, dinv, w2, b1)


def _tc3_body(aggA_ref, aggB_ref, ht2_ref, dinv_ref, bidx_ref, b2_ref,
              wl_ref, bl_ref, out_ref, psum_s):
  i = pl.program_id(0)

  @pl.when(i == 0)
  def _():
    psum_s[...] = jnp.zeros_like(psum_s)

  agg = aggA_ref[...] + aggB_ref[...] + ht2_ref[...]
  out2 = dinv_ref[...] * agg + b2_ref[...]
  onehot = (bidx_ref[...] == lax.broadcasted_iota(jnp.int32, (_R, G), 1))
  onehot = onehot.astype(jnp.float32)
  aug = jnp.concatenate([out2, jnp.ones((_R, 1), jnp.float32)], axis=1)
  psum_s[...] += lax.dot_general(onehot, aug, (((0,), (0,)), ((), ())),
                                 precision=_HI,
                                 preferred_element_type=jnp.float32)

  @pl.when(i == _GRID - 1)
  def _():
    acc = psum_s[...]
    pooled = acc[:, :DH] / jnp.maximum(acc[:, DH:], 1.0)
    out_ref[...] = jnp.dot(pooled, wl_ref[...], precision=_HI,
                           preferred_element_type=jnp.float32) + bl_ref[...]


def _tc3(aggA, aggB, ht2, dinv, bidx, b2, wl, bl):
  return pl.pallas_call(
      _tc3_body,
      grid=(_GRID,),
      in_specs=[
          pl.BlockSpec((_R, DH), lambda i: (i, 0)),
          pl.BlockSpec((_R, DH), lambda i: (i, 0)),
          pl.BlockSpec((_R, DH), lambda i: (i, 0)),
          pl.BlockSpec((_R, 1), lambda i: (i, 0)),
          pl.BlockSpec((_R, 1), lambda i: (i, 0)),
          pl.BlockSpec((1, DH), lambda i: (0, 0)),
          pl.BlockSpec((DH, DOUT), lambda i: (0, 0)),
          pl.BlockSpec((1, DOUT), lambda i: (0, 0)),
      ],
      out_specs=pl.BlockSpec((G, DOUT), lambda i: (0, 0)),
      out_shape=jax.ShapeDtypeStruct((G, DOUT), jnp.float32),
      scratch_shapes=[pltpu.VMEM((G, DH + 1), jnp.float32)],
  )(aggA, aggB, ht2, dinv, bidx, b2, wl, bl)


def kernel(inputs, edge_index, batch_indexes, W1, b1, W2, b2, W_lin, b_lin):
  src_pad = (jnp.arange(EPAD, dtype=jnp.int32) * 137) % N
  dst_pad = N + (jnp.arange(EPAD, dtype=jnp.int32) % (NPAD - N))
  src2d = jnp.concatenate([edge_index[0], src_pad]).reshape(NCHUNKP, CHUNK)
  dst2d = jnp.concatenate([edge_index[1], dst_pad]).reshape(NCHUNKP, CHUNK)
  zeros = jnp.zeros((NPAD, DH), jnp.float32)

  deg = _deg(dst2d)
  degA = deg[:N].reshape(N, 1)
  degB = deg[NPAD:NPAD + N].reshape(N, 1)

  ht1, dinv = _tc1(inputs, W1, degA, degB)
  agg1 = _agg(ht1, src2d, dst2d, zeros)
  ht2 = _tc2(agg1[:N], agg1[NPAD:NPAD + N], ht1, dinv, W2, b1.reshape(1, DH))
  agg2 = _agg(ht2, src2d, dst2d, zeros)
  return _tc3(agg2[:N], agg2[NPAD:NPAD + N], ht2, dinv,
              batch_indexes.reshape(N, 1), b2.reshape(1, DH), W_lin,
              b_lin.reshape(1, DOUT))


# speedup vs baseline: 1.1874x; 1.1874x over previous
import functools

import jax
import jax.numpy as jnp
from jax import lax
from jax.experimental import pallas as pl
from jax.experimental.pallas import tpu as pltpu
from jax.experimental.pallas import tpu_sc as plsc

N = 10000
E = 320000
DIN = 128
DH = 64
DOUT = 10
G = 64

NC, NS = 2, 16
NW = NC * NS
CHUNK = 128
CPT = 80
NCHUNKP = NW * CPT
EPAD = NCHUNKP * CHUNK - E
NPAD = 10240
DSLICE = NPAD // NS
RSLICE = NPAD // NS
NB = 8

_mesh = plsc.VectorSubcoreMesh(core_axis_name="c", subcore_axis_name="s")
_sc_params = pltpu.CompilerParams(use_tc_tiling_on_sc=False)


def _wid(cid, sid):
  return sid * NC + cid


@functools.partial(
    pl.kernel,
    out_type=jax.ShapeDtypeStruct((NC * NPAD,), jnp.float32),
    mesh=_mesh,
    scratch_types=[
        pltpu.VMEM((CPT, CHUNK), jnp.int32),
        pltpu.VMEM((CHUNK,), jnp.float32),
        pltpu.VMEM((DSLICE,), jnp.float32),
        pltpu.VMEM_SHARED((NPAD,), jnp.float32),
        pltpu.SemaphoreType.DMA,
    ],
    compiler_params=_sc_params,
)
def _deg(dst_hbm, out_hbm, didx_v, ones_v, zbuf_v, acc_sh, sem):
  cid = lax.axis_index("c")
  sid = lax.axis_index("s")
  wid = _wid(cid, sid)
  pltpu.sync_copy(dst_hbm.at[pl.ds(wid * CPT, CPT)], didx_v)
  for j in range(CHUNK // 16):
    ones_v[pl.ds(j * 16, 16)] = jnp.ones((16,), jnp.float32)
  for j in range(DSLICE // 16):
    zbuf_v[pl.ds(j * 16, 16)] = jnp.zeros((16,), jnp.float32)
  pltpu.sync_copy(zbuf_v, acc_sh.at[pl.ds(sid * DSLICE, DSLICE)])
  plsc.subcore_barrier()

  K = 20
  def group(g, carry):
    def fire(j, c):
      pltpu.async_copy(ones_v, acc_sh.at[didx_v.at[g * K + j]], sem, add=True)
      return c

    lax.fori_loop(0, K, fire, 0)

    def drain(j, c):
      pltpu.make_async_copy(ones_v, acc_sh.at[didx_v.at[g * K + j]],
                            sem).wait()
      return c

    lax.fori_loop(0, K, drain, 0)
    return carry

  lax.fori_loop(0, CPT // K, group, 0)
  plsc.subcore_barrier()
  pltpu.sync_copy(
      acc_sh.at[pl.ds(sid * DSLICE, DSLICE)],
      out_hbm.at[pl.ds(cid * NPAD + sid * DSLICE, DSLICE)],
  )


@functools.partial(
    pl.kernel,
    out_type=jax.ShapeDtypeStruct((NC * NPAD, DH), jnp.float32),
    mesh=_mesh,
    scratch_types=[
    ] + [pltpu.VMEM((CHUNK,), jnp.int32)] * (2 * NB)
      + [pltpu.VMEM((CHUNK, DH), jnp.float32)] * NB + [
        pltpu.VMEM_SHARED((NPAD, DH), jnp.float32),
    ] + [pltpu.SemaphoreType.DMA] * (3 * NB),
    compiler_params=_sc_params,
)
def _agg(h_hbm, src_hbm, dst_hbm, zero_hbm, out_hbm, *rest):
  sidx = rest[:NB]
  didx = rest[NB:2 * NB]
  rows = rest[2 * NB:3 * NB]
  acc_sh = rest[3 * NB]
  isem = rest[3 * NB + 1:3 * NB + 1 + NB]
  gsem = rest[3 * NB + 1 + NB:3 * NB + 1 + 2 * NB]
  ssem = rest[3 * NB + 1 + 2 * NB:]
  cid = lax.axis_index("c")
  sid = lax.axis_index("s")
  wid = _wid(cid, sid)
  base = wid * CPT
  pltpu.sync_copy(zero_hbm.at[pl.ds(sid * RSLICE, RSLICE)],
                  acc_sh.at[pl.ds(sid * RSLICE, RSLICE)])
  plsc.subcore_barrier()

  def i_fire(j, b):
    c = wid + j * NW
    pltpu.async_copy(src_hbm.at[c], sidx[b], isem[b])
    pltpu.async_copy(dst_hbm.at[c], didx[b], isem[b])

  def i_wait(j, b):
    c = wid + j * NW
    pltpu.make_async_copy(src_hbm.at[c], sidx[b], isem[b]).wait()
    pltpu.make_async_copy(dst_hbm.at[c], didx[b], isem[b]).wait()

  def g_fire(b):
    pltpu.async_copy(h_hbm.at[sidx[b]], rows[b], gsem[b])

  def g_wait(b):
    pltpu.make_async_copy(h_hbm.at[sidx[b]], rows[b], gsem[b]).wait()

  def s_fire(b):
    pltpu.async_copy(rows[b], acc_sh.at[didx[b]], ssem[b], add=True)

  def s_wait(b):
    pltpu.make_async_copy(rows[b], acc_sh.at[didx[b]], ssem[b]).wait()

  for j0 in range(4):
    i_fire(j0, j0)
  i_wait(0, 0)
  g_fire(0)
  i_wait(1, 1)
  g_fire(1)

  def group(g, carry):
    for b in range(NB):
      j = g * NB + b
      b2 = (b + 2) % NB
      b4 = (b + 4) % NB
      g_wait(b)
      s_fire(b)
      j4 = j + 4

      @pl.when(j4 < CPT)
      def _():
        @pl.when(j4 >= NB)
        def _():
          s_wait(b4)
        i_fire(j4, b4)

      j2 = j + 2

      @pl.when(j2 < CPT)
      def _():
        i_wait(j2, b2)
        g_fire(b2)

    return carry

  lax.fori_loop(0, CPT // NB, group, 0)
  for b in range(NB):
    s_wait(b)
  plsc.subcore_barrier()
  pltpu.sync_copy(acc_sh.at[pl.ds(sid * RSLICE, RSLICE)],
                  out_hbm.at[pl.ds(cid * NPAD + sid * RSLICE, RSLICE)])


_R = 1000
_GRID = N // _R

_HI = lax.Precision.HIGHEST


def _tc1_body(x_ref, w1_ref, degA_ref, degB_ref, ht_ref, dinv_ref):
  deg = degA_ref[...] + degB_ref[...] + 1.0
  dinv = lax.rsqrt(deg)
  dinv_ref[...] = dinv
  h = jnp.dot(x_ref[...], w1_ref[...], precision=_HI,
              preferred_element_type=jnp.float32)
  ht_ref[...] = h * dinv


def _tc1(x, w1, degA, degB):
  return pl.pallas_call(
      _tc1_body,
      grid=(_GRID,),
      in_specs=[
          pl.BlockSpec((_R, DIN), lambda i: (i, 0)),
          pl.BlockSpec((DIN, DH), lambda i: (0, 0)),
          pl.BlockSpec((_R, 1), lambda i: (i, 0)),
          pl.BlockSpec((_R, 1), lambda i: (i, 0)),
      ],
      out_specs=[
          pl.BlockSpec((_R, DH), lambda i: (i, 0)),
          pl.BlockSpec((_R, 1), lambda i: (i, 0)),
      ],
      out_shape=[
          jax.ShapeDtypeStruct((N, DH), jnp.float32),
          jax.ShapeDtypeStruct((N, 1), jnp.float32),
      ],
  )(x, w1, degA, degB)


def _tc2_body(aggA_ref, aggB_ref, ht1_ref, dinv_ref, w2_ref, b1_ref, ht2_ref):
  agg = aggA_ref[...] + aggB_ref[...] + ht1_ref[...]
  out1 = dinv_ref[...] * agg + b1_ref[...]
  h2 = jnp.dot(out1, w2_ref[...], precision=_HI,
               preferred_element_type=jnp.float32)
  ht2_ref[...] = dinv_ref[...] * h2


def _tc2(aggA, aggB, ht1, dinv, w2, b1):
  return pl.pallas_call(
      _tc2_body,
      grid=(_GRID,),
      in_specs=[
          pl.BlockSpec((_R, DH), lambda i: (i, 0)),
          pl.BlockSpec((_R, DH), lambda i: (i, 0)),
          pl.BlockSpec((_R, DH), lambda i: (i, 0)),
          pl.BlockSpec((_R, 1), lambda i: (i, 0)),
          pl.BlockSpec((DH, DH), lambda i: (0, 0)),
          pl.BlockSpec((1, DH), lambda i: (0, 0)),
      ],
      out_specs=pl.BlockSpec((_R, DH), lambda i: (i, 0)),
      out_shape=jax.ShapeDtypeStruct((N, DH), jnp.float32),
  )(aggA, aggB, ht1, dinv, w2, b1)


def _tc3_body(aggA_ref, aggB_ref, ht2_ref, dinv_ref, bidx_ref, b2_ref,
              wl_ref, bl_ref, out_ref, psum_s):
  i = pl.program_id(0)

  @pl.when(i == 0)
  def _():
    psum_s[...] = jnp.zeros_like(psum_s)

  agg = aggA_ref[...] + aggB_ref[...] + ht2_ref[...]
  out2 = dinv_ref[...] * agg + b2_ref[...]
  onehot = (bidx_ref[...] == lax.broadcasted_iota(jnp.int32, (_R, G), 1))
  onehot = onehot.astype(jnp.float32)
  aug = jnp.concatenate([out2, jnp.ones((_R, 1), jnp.float32)], axis=1)
  psum_s[...] += lax.dot_general(onehot, aug, (((0,), (0,)), ((), ())),
                                 precision=_HI,
                                 preferred_element_type=jnp.float32)

  @pl.when(i == _GRID - 1)
  def _():
    acc = psum_s[...]
    pooled = acc[:, :DH] / jnp.maximum(acc[:, DH:], 1.0)
    out_ref[...] = jnp.dot(pooled, wl_ref[...], precision=_HI,
                           preferred_element_type=jnp.float32) + bl_ref[...]


def _tc3(aggA, aggB, ht2, dinv, bidx, b2, wl, bl):
  return pl.pallas_call(
      _tc3_body,
      grid=(_GRID,),
      in_specs=[
          pl.BlockSpec((_R, DH), lambda i: (i, 0)),
          pl.BlockSpec((_R, DH), lambda i: (i, 0)),
          pl.BlockSpec((_R, DH), lambda i: (i, 0)),
          pl.BlockSpec((_R, 1), lambda i: (i, 0)),
          pl.BlockSpec((_R, 1), lambda i: (i, 0)),
          pl.BlockSpec((1, DH), lambda i: (0, 0)),
          pl.BlockSpec((DH, DOUT), lambda i: (0, 0)),
          pl.BlockSpec((1, DOUT), lambda i: (0, 0)),
      ],
      out_specs=pl.BlockSpec((G, DOUT), lambda i: (0, 0)),
      out_shape=jax.ShapeDtypeStruct((G, DOUT), jnp.float32),
      scratch_shapes=[pltpu.VMEM((G, DH + 1), jnp.float32)],
  )(aggA, aggB, ht2, dinv, bidx, b2, wl, bl)


def kernel(inputs, edge_index, batch_indexes, W1, b1, W2, b2, W_lin, b_lin):
  src_pad = (jnp.arange(EPAD, dtype=jnp.int32) * 137) % N
  dst_pad = N + (jnp.arange(EPAD, dtype=jnp.int32) % (NPAD - N))
  src2d = jnp.concatenate([edge_index[0], src_pad]).reshape(NCHUNKP, CHUNK)
  dst2d = jnp.concatenate([edge_index[1], dst_pad]).reshape(NCHUNKP, CHUNK)
  zeros = jnp.zeros((NPAD, DH), jnp.float32)

  deg = _deg(dst2d)
  degA = deg[:N].reshape(N, 1)
  degB = deg[NPAD:NPAD + N].reshape(N, 1)

  ht1, dinv = _tc1(inputs, W1, degA, degB)
  agg1 = _agg(ht1, src2d, dst2d, zeros)
  ht2 = _tc2(agg1[:N], agg1[NPAD:NPAD + N], ht1, dinv, W2, b1.reshape(1, DH))
  agg2 = _agg(ht2, src2d, dst2d, zeros)
  return _tc3(agg2[:N], agg2[NPAD:NPAD + N], ht2, dinv,
              batch_indexes.reshape(N, 1), b2.reshape(1, DH), W_lin,
              b_lin.reshape(1, DOUT))


# speedup vs baseline: 1.2380x; 1.0426x over previous
import functools

import jax
import jax.numpy as jnp
from jax import lax
from jax.experimental import pallas as pl
from jax.experimental.pallas import tpu as pltpu
from jax.experimental.pallas import tpu_sc as plsc

N = 10000
E = 320000
DIN = 128
DH = 64
DOUT = 10
G = 64

NC, NS = 2, 16
NW = NC * NS
CHUNK = 128
CPT = 80
NCHUNKP = NW * CPT
EPAD = NCHUNKP * CHUNK - E
NPAD = 10240
DSLICE = NPAD // NS
RSLICE = NPAD // NS
NB = 10

_mesh = plsc.VectorSubcoreMesh(core_axis_name="c", subcore_axis_name="s")
_sc_params = pltpu.CompilerParams(use_tc_tiling_on_sc=False)


def _wid(cid, sid):
  return sid * NC + cid


@functools.partial(
    pl.kernel,
    out_type=jax.ShapeDtypeStruct((NC * NPAD,), jnp.float32),
    mesh=_mesh,
    scratch_types=[
        pltpu.VMEM((CPT, CHUNK), jnp.int32),
        pltpu.VMEM((CHUNK,), jnp.float32),
        pltpu.VMEM((DSLICE,), jnp.float32),
        pltpu.VMEM_SHARED((NPAD,), jnp.float32),
        pltpu.SemaphoreType.DMA,
    ],
    compiler_params=_sc_params,
)
def _deg(dst_hbm, out_hbm, didx_v, ones_v, zbuf_v, acc_sh, sem):
  cid = lax.axis_index("c")
  sid = lax.axis_index("s")
  wid = _wid(cid, sid)
  pltpu.sync_copy(dst_hbm.at[pl.ds(wid * CPT, CPT)], didx_v)
  for j in range(CHUNK // 16):
    ones_v[pl.ds(j * 16, 16)] = jnp.ones((16,), jnp.float32)
  for j in range(DSLICE // 16):
    zbuf_v[pl.ds(j * 16, 16)] = jnp.zeros((16,), jnp.float32)
  pltpu.sync_copy(zbuf_v, acc_sh.at[pl.ds(sid * DSLICE, DSLICE)])
  plsc.subcore_barrier()

  K = 20
  def group(g, carry):
    def fire(j, c):
      pltpu.async_copy(ones_v, acc_sh.at[didx_v.at[g * K + j]], sem, add=True)
      return c

    lax.fori_loop(0, K, fire, 0)

    def drain(j, c):
      pltpu.make_async_copy(ones_v, acc_sh.at[didx_v.at[g * K + j]],
                            sem).wait()
      return c

    lax.fori_loop(0, K, drain, 0)
    return carry

  lax.fori_loop(0, CPT // K, group, 0)
  plsc.subcore_barrier()
  pltpu.sync_copy(
      acc_sh.at[pl.ds(sid * DSLICE, DSLICE)],
      out_hbm.at[pl.ds(cid * NPAD + sid * DSLICE, DSLICE)],
  )


@functools.partial(
    pl.kernel,
    out_type=jax.ShapeDtypeStruct((NC * NPAD, DH), jnp.float32),
    mesh=_mesh,
    scratch_types=[
    ] + [pltpu.VMEM((CHUNK,), jnp.int32)] * (2 * NB)
      + [pltpu.VMEM((CHUNK, DH), jnp.float32)] * NB + [
        pltpu.VMEM_SHARED((NPAD, DH), jnp.float32),
    ] + [pltpu.SemaphoreType.DMA] * (3 * NB),
    compiler_params=_sc_params,
)
def _agg(h_hbm, src_hbm, dst_hbm, zero_hbm, out_hbm, *rest):
  sidx = rest[:NB]
  didx = rest[NB:2 * NB]
  rows = rest[2 * NB:3 * NB]
  acc_sh = rest[3 * NB]
  isem = rest[3 * NB + 1:3 * NB + 1 + NB]
  gsem = rest[3 * NB + 1 + NB:3 * NB + 1 + 2 * NB]
  ssem = rest[3 * NB + 1 + 2 * NB:]
  cid = lax.axis_index("c")
  sid = lax.axis_index("s")
  wid = _wid(cid, sid)
  base = wid * CPT
  pltpu.sync_copy(zero_hbm.at[pl.ds(sid * RSLICE, RSLICE)],
                  acc_sh.at[pl.ds(sid * RSLICE, RSLICE)])
  plsc.subcore_barrier()

  def i_fire(j, b):
    c = wid + j * NW
    pltpu.async_copy(src_hbm.at[c], sidx[b], isem[b])
    pltpu.async_copy(dst_hbm.at[c], didx[b], isem[b])

  def i_wait(j, b):
    c = wid + j * NW
    pltpu.make_async_copy(src_hbm.at[c], sidx[b], isem[b]).wait()
    pltpu.make_async_copy(dst_hbm.at[c], didx[b], isem[b]).wait()

  def g_fire(b):
    pltpu.async_copy(h_hbm.at[sidx[b]], rows[b], gsem[b])

  def g_wait(b):
    pltpu.make_async_copy(h_hbm.at[sidx[b]], rows[b], gsem[b]).wait()

  def s_fire(b):
    pltpu.async_copy(rows[b], acc_sh.at[didx[b]], ssem[b], add=True)

  def s_wait(b):
    pltpu.make_async_copy(rows[b], acc_sh.at[didx[b]], ssem[b]).wait()

  for j0 in range(6):
    i_fire(j0, j0)
  for j0 in range(3):
    i_wait(j0, j0)
    g_fire(j0)

  def group(g, carry):
    for b in range(NB):
      j = g * NB + b
      b3 = (b + 3) % NB
      b6 = (b + 6) % NB
      g_wait(b)
      s_fire(b)
      j6 = j + 6

      @pl.when(j6 < CPT)
      def _():
        @pl.when(j6 >= NB)
        def _():
          s_wait(b6)
        i_fire(j6, b6)

      j3 = j + 3

      @pl.when(j3 < CPT)
      def _():
        i_wait(j3, b3)
        g_fire(b3)

    return carry

  lax.fori_loop(0, CPT // NB, group, 0)
  for b in range(NB):
    s_wait(b)
  plsc.subcore_barrier()
  pltpu.sync_copy(acc_sh.at[pl.ds(sid * RSLICE, RSLICE)],
                  out_hbm.at[pl.ds(cid * NPAD + sid * RSLICE, RSLICE)])


_R = 1000
_GRID = N // _R

_HI = lax.Precision.HIGHEST


def _tc1_body(x_ref, w1_ref, degA_ref, degB_ref, ht_ref, dinv_ref):
  deg = degA_ref[...] + degB_ref[...] + 1.0
  dinv = lax.rsqrt(deg)
  dinv_ref[...] = dinv
  h = jnp.dot(x_ref[...], w1_ref[...], precision=_HI,
              preferred_element_type=jnp.float32)
  ht_ref[...] = h * dinv


def _tc1(x, w1, degA, degB):
  return pl.pallas_call(
      _tc1_body,
      grid=(_GRID,),
      in_specs=[
          pl.BlockSpec((_R, DIN), lambda i: (i, 0)),
          pl.BlockSpec((DIN, DH), lambda i: (0, 0)),
          pl.BlockSpec((_R, 1), lambda i: (i, 0)),
          pl.BlockSpec((_R, 1), lambda i: (i, 0)),
      ],
      out_specs=[
          pl.BlockSpec((_R, DH), lambda i: (i, 0)),
          pl.BlockSpec((_R, 1), lambda i: (i, 0)),
      ],
      out_shape=[
          jax.ShapeDtypeStruct((N, DH), jnp.float32),
          jax.ShapeDtypeStruct((N, 1), jnp.float32),
      ],
  )(x, w1, degA, degB)


def _tc2_body(aggA_ref, aggB_ref, ht1_ref, dinv_ref, w2_ref, b1_ref, ht2_ref):
  agg = aggA_ref[...] + aggB_ref[...] + ht1_ref[...]
  out1 = dinv_ref[...] * agg + b1_ref[...]
  h2 = jnp.dot(out1, w2_ref[...], precision=_HI,
               preferred_element_type=jnp.float32)
  ht2_ref[...] = dinv_ref[...] * h2


def _tc2(aggA, aggB, ht1, dinv, w2, b1):
  return pl.pallas_call(
      _tc2_body,
      grid=(_GRID,),
      in_specs=[
          pl.BlockSpec((_R, DH), lambda i: (i, 0)),
          pl.BlockSpec((_R, DH), lambda i: (i, 0)),
          pl.BlockSpec((_R, DH), lambda i: (i, 0)),
          pl.BlockSpec((_R, 1), lambda i: (i, 0)),
          pl.BlockSpec((DH, DH), lambda i: (0, 0)),
          pl.BlockSpec((1, DH), lambda i: (0, 0)),
      ],
      out_specs=pl.BlockSpec((_R, DH), lambda i: (i, 0)),
      out_shape=jax.ShapeDtypeStruct((N, DH), jnp.float32),
  )(aggA, aggB, ht1, dinv, w2, b1)


def _tc3_body(aggA_ref, aggB_ref, ht2_ref, dinv_ref, bidx_ref, b2_ref,
              wl_ref, bl_ref, out_ref, psum_s):
  i = pl.program_id(0)

  @pl.when(i == 0)
  def _():
    psum_s[...] = jnp.zeros_like(psum_s)

  agg = aggA_ref[...] + aggB_ref[...] + ht2_ref[...]
  out2 = dinv_ref[...] * agg + b2_ref[...]
  onehot = (bidx_ref[...] == lax.broadcasted_iota(jnp.int32, (_R, G), 1))
  onehot = onehot.astype(jnp.float32)
  aug = jnp.concatenate([out2, jnp.ones((_R, 1), jnp.float32)], axis=1)
  psum_s[...] += lax.dot_general(onehot, aug, (((0,), (0,)), ((), ())),
                                 precision=_HI,
                                 preferred_element_type=jnp.float32)

  @pl.when(i == _GRID - 1)
  def _():
    acc = psum_s[...]
    pooled = acc[:, :DH] / jnp.maximum(acc[:, DH:], 1.0)
    out_ref[...] = jnp.dot(pooled, wl_ref[...], precision=_HI,
                           preferred_element_type=jnp.float32) + bl_ref[...]


def _tc3(aggA, aggB, ht2, dinv, bidx, b2, wl, bl):
  return pl.pallas_call(
      _tc3_body,
      grid=(_GRID,),
      in_specs=[
          pl.BlockSpec((_R, DH), lambda i: (i, 0)),
          pl.BlockSpec((_R, DH), lambda i: (i, 0)),
          pl.BlockSpec((_R, DH), lambda i: (i, 0)),
          pl.BlockSpec((_R, 1), lambda i: (i, 0)),
          pl.BlockSpec((_R, 1), lambda i: (i, 0)),
          pl.BlockSpec((1, DH), lambda i: (0, 0)),
          pl.BlockSpec((DH, DOUT), lambda i: (0, 0)),
          pl.BlockSpec((1, DOUT), lambda i: (0, 0)),
      ],
      out_specs=pl.BlockSpec((G, DOUT), lambda i: (0, 0)),
      out_shape=jax.ShapeDtypeStruct((G, DOUT), jnp.float32),
      scratch_shapes=[pltpu.VMEM((G, DH + 1), jnp.float32)],
  )(aggA, aggB, ht2, dinv, bidx, b2, wl, bl)


def kernel(inputs, edge_index, batch_indexes, W1, b1, W2, b2, W_lin, b_lin):
  src_pad = (jnp.arange(EPAD, dtype=jnp.int32) * 137) % N
  dst_pad = N + (jnp.arange(EPAD, dtype=jnp.int32) % (NPAD - N))
  src2d = jnp.concatenate([edge_index[0], src_pad]).reshape(NCHUNKP, CHUNK)
  dst2d = jnp.concatenate([edge_index[1], dst_pad]).reshape(NCHUNKP, CHUNK)
  zeros = jnp.zeros((NPAD, DH), jnp.float32)

  deg = _deg(dst2d)
  degA = deg[:N].reshape(N, 1)
  degB = deg[NPAD:NPAD + N].reshape(N, 1)

  ht1, dinv = _tc1(inputs, W1, degA, degB)
  agg1 = _agg(ht1, src2d, dst2d, zeros)
  ht2 = _tc2(agg1[:N], agg1[NPAD:NPAD + N], ht1, dinv, W2, b1.reshape(1, DH))
  agg2 = _agg(ht2, src2d, dst2d, zeros)
  return _tc3(agg2[:N], agg2[NPAD:NPAD + N], ht2, dinv,
              batch_indexes.reshape(N, 1), b2.reshape(1, DH), W_lin,
              b_lin.reshape(1, DOUT))
